# baseline (speedup 1.0000x reference)
import jax
import jax.numpy as jnp
from jax.experimental import pallas as pl
from jax.experimental.pallas import tpu as pltpu

EPS = 1e-5
IN_FEATURES = 8
HIDDEN = 64


def _round_up(n, m):
    return (n + m - 1) // m * m


def _make_body(batch, tile_b, needs_mask):
    inv_b = 1.0 / float(batch)

    def body(x_ref, w1a_ref, w2_ref, b2_ref, w3_ref, b3_ref, o_ref,
             m1_ref, r1_ref, m2_ref, r2_ref,
             w2p_ref, b2p_ref, w3p_ref, b3p_ref):
        ph = pl.program_id(0)
        t = pl.program_id(1)
        last = pl.num_programs(1) - 1

        def layer1():
            xb = x_ref[:, pl.ds(t * tile_b, tile_b)].astype(jnp.bfloat16)
            z = jnp.dot(w1a_ref[...], xb, preferred_element_type=jnp.float32)
            return jnp.maximum(z, 0.0)

        def layer2():
            xb = x_ref[:, pl.ds(t * tile_b, tile_b)].astype(jnp.bfloat16)
            z1 = jnp.dot(w1a_ref[...], xb, preferred_element_type=jnp.float32)
            h1b = jnp.maximum(z1.astype(jnp.bfloat16), jnp.bfloat16(0.0))
            z = jnp.dot(w2p_ref[...], h1b, preferred_element_type=jnp.float32)
            return jnp.maximum(z + b2p_ref[...], 0.0)

        def accumulate(h, sum_ref, sq_ref):
            if needs_mask:
                col = (jax.lax.broadcasted_iota(jnp.int32, (1, tile_b), 1)
                       + t * tile_b)
                h = h * (col < batch).astype(jnp.float32)
            s = jnp.sum(h, axis=1, keepdims=True)
            sq = jnp.sum(h * h, axis=1, keepdims=True)

            @pl.when(t == 0)
            def _():
                sum_ref[...] = s
                sq_ref[...] = sq

            @pl.when(t > 0)
            def _():
                sum_ref[...] += s
                sq_ref[...] += sq

            @pl.when(t == last)
            def _():
                mean = sum_ref[...] * inv_b
                var = sq_ref[...] * inv_b - mean * mean
                sum_ref[...] = mean
                sq_ref[...] = jax.lax.rsqrt(var + EPS)

        @pl.when(ph == 0)
        def _():
            accumulate(layer1(), m1_ref, r1_ref)

            @pl.when(t == last)
            def _():
                r1 = r1_ref[...]
                r1_row = r1.reshape(1, HIDDEN)
                w2 = w2_ref[...]
                w2p_ref[...] = (w2 * r1_row).astype(jnp.bfloat16)
                b2p_ref[...] = b2_ref[...] - jnp.dot(
                    w2, m1_ref[...] * r1, preferred_element_type=jnp.float32)

        @pl.when(ph == 1)
        def _():
            accumulate(layer2(), m2_ref, r2_ref)

            @pl.when(t == last)
            def _():
                r2 = r2_ref[...]
                w3 = w3_ref[...]
                w3p_ref[...] = w3 * r2.reshape(1, HIDDEN)
                b3p_ref[...] = b3_ref[...] - jnp.dot(
                    w3, m2_ref[...] * r2, preferred_element_type=jnp.float32)

        @pl.when(ph == 2)
        def _():
            h2 = layer2()
            out = jnp.dot(w3p_ref[...], h2,
                          preferred_element_type=jnp.float32) + b3p_ref[...]
            o_ref[:, pl.ds(t * tile_b, tile_b)] = out

    return body


def kernel(x, w1, b1, w2, b2, w3, b3, *, block_b=65536):
    B, f_in = x.shape
    assert f_in == IN_FEATURES
    assert B > 1

    tile_b = min(_round_up(block_b, 128), _round_up(B, 128))
    padded_b = _round_up(B, tile_b)
    num_tiles = padded_b // tile_b
    needs_mask = padded_b != B

    x_fm = jnp.concatenate(
        [x.astype(jnp.float32).T, jnp.ones((1, B), jnp.float32)], axis=0)
    if needs_mask:
        x_fm = jnp.pad(x_fm, ((0, 0), (0, padded_b - B)))
    w1a = jnp.concatenate([w1, b1], axis=1).astype(jnp.bfloat16)

    def const(ph, t):
        return (0, 0)

    grid_spec = pltpu.PrefetchScalarGridSpec(
        num_scalar_prefetch=0,
        grid=(3, num_tiles),
        in_specs=[
            pl.BlockSpec((IN_FEATURES + 1, padded_b), const),
            pl.BlockSpec((HIDDEN, IN_FEATURES + 1), const),
            pl.BlockSpec((HIDDEN, HIDDEN), const),
            pl.BlockSpec((HIDDEN, 1), const),
            pl.BlockSpec((1, HIDDEN), const),
            pl.BlockSpec((1, 1), const),
        ],
        out_specs=pl.BlockSpec((1, padded_b), const),
        scratch_shapes=[
            pltpu.VMEM((HIDDEN, 1), jnp.float32),
            pltpu.VMEM((HIDDEN, 1), jnp.float32),
            pltpu.VMEM((HIDDEN, 1), jnp.float32),
            pltpu.VMEM((HIDDEN, 1), jnp.float32),
            pltpu.VMEM((HIDDEN, HIDDEN), jnp.bfloat16),
            pltpu.VMEM((HIDDEN, 1), jnp.float32),
            pltpu.VMEM((1, HIDDEN), jnp.float32),
            pltpu.VMEM((1, 1), jnp.float32),
        ],
    )

    out_fm = pl.pallas_call(
        _make_body(B, tile_b, needs_mask),
        out_shape=jax.ShapeDtypeStruct((1, padded_b), jnp.float32),
        grid_spec=grid_spec,
        compiler_params=pltpu.CompilerParams(
            dimension_semantics=("arbitrary", "arbitrary")),
    )(x_fm, w1a, w2, b2, w3, b3)

    return out_fm[:, :B].T


# speedup vs baseline: 1.1727x; 1.1727x over previous
import jax
import jax.numpy as jnp
from jax.experimental import pallas as pl
from jax.experimental.pallas import tpu as pltpu

EPS = 1e-5
IN_FEATURES = 8
HIDDEN = 64


def _round_up(n, m):
    return (n + m - 1) // m * m


def _make_body(batch, tile_b, needs_mask):
    inv_b = 1.0 / float(batch)

    def body(x_ref, w1a_ref, w2_ref, b2_ref, w3_ref, b3_ref, o_ref,
             m1_ref, r1_ref, m2_ref, r2_ref,
             w2p_ref, b2p_ref, w3p_ref, b3p_ref):
        ph = pl.program_id(0)
        t = pl.program_id(1)
        last = pl.num_programs(1) - 1

        def layer1():
            xb = x_ref[:, pl.ds(t * tile_b, tile_b)]
            z = jnp.dot(w1a_ref[...], xb, preferred_element_type=jnp.float32)
            return jnp.maximum(z, 0.0)

        def layer2():
            h1b = layer1().astype(jnp.bfloat16)
            z = jnp.dot(w2p_ref[...], h1b, preferred_element_type=jnp.float32)
            return jnp.maximum(z + b2p_ref[...], 0.0)

        def accumulate(h, sum_ref, sq_ref):
            if needs_mask:
                col = (jax.lax.broadcasted_iota(jnp.int32, (1, tile_b), 1)
                       + t * tile_b)
                h = h * (col < batch).astype(jnp.float32)
            s = jnp.sum(h, axis=1, keepdims=True)
            sq = jnp.sum(h * h, axis=1, keepdims=True)

            @pl.when(t == 0)
            def _():
                sum_ref[...] = s
                sq_ref[...] = sq

            @pl.when(t > 0)
            def _():
                sum_ref[...] += s
                sq_ref[...] += sq

            @pl.when(t == last)
            def _():
                mean = sum_ref[...] * inv_b
                var = sq_ref[...] * inv_b - mean * mean
                sum_ref[...] = mean
                sq_ref[...] = jax.lax.rsqrt(var + EPS)

        @pl.when(ph == 0)
        def _():
            accumulate(layer1(), m1_ref, r1_ref)

            @pl.when(t == last)
            def _():
                r1 = r1_ref[...]
                r1_row = r1.reshape(1, HIDDEN)
                w2 = w2_ref[...]
                w2p_ref[...] = (w2 * r1_row).astype(jnp.bfloat16)
                b2p_ref[...] = b2_ref[...] - jnp.dot(
                    w2, m1_ref[...] * r1, preferred_element_type=jnp.float32)

        @pl.when(ph == 1)
        def _():
            accumulate(layer2(), m2_ref, r2_ref)

            @pl.when(t == last)
            def _():
                r2 = r2_ref[...]
                w3 = w3_ref[...]
                w3p_ref[...] = w3 * r2.reshape(1, HIDDEN)
                b3p_ref[...] = b3_ref[...] - jnp.dot(
                    w3, m2_ref[...] * r2, preferred_element_type=jnp.float32)

        @pl.when(ph == 2)
        def _():
            h2 = layer2()
            out = jnp.dot(w3p_ref[...], h2,
                          preferred_element_type=jnp.float32) + b3p_ref[...]
            o_ref[:, pl.ds(t * tile_b, tile_b)] = out

    return body


def kernel(x, w1, b1, w2, b2, w3, b3, *, block_b=131072):
    B, f_in = x.shape
    assert f_in == IN_FEATURES
    assert B > 1

    tile_b = min(_round_up(block_b, 128), _round_up(B, 128))
    padded_b = _round_up(B, tile_b)
    num_tiles = padded_b // tile_b
    needs_mask = padded_b != B

    x_fm = jnp.concatenate(
        [x.astype(jnp.float32).T, jnp.ones((1, B), jnp.float32)],
        axis=0).astype(jnp.bfloat16)
    if needs_mask:
        x_fm = jnp.pad(x_fm, ((0, 0), (0, padded_b - B)))
    w1a = jnp.concatenate([w1, b1], axis=1).astype(jnp.bfloat16)

    def const(ph, t):
        return (0, 0)

    grid_spec = pltpu.PrefetchScalarGridSpec(
        num_scalar_prefetch=0,
        grid=(3, num_tiles),
        in_specs=[
            pl.BlockSpec((IN_FEATURES + 1, padded_b), const),
            pl.BlockSpec((HIDDEN, IN_FEATURES + 1), const),
            pl.BlockSpec((HIDDEN, HIDDEN), const),
            pl.BlockSpec((HIDDEN, 1), const),
            pl.BlockSpec((1, HIDDEN), const),
            pl.BlockSpec((1, 1), const),
        ],
        out_specs=pl.BlockSpec((1, padded_b), const),
        scratch_shapes=[
            pltpu.VMEM((HIDDEN, 1), jnp.float32),
            pltpu.VMEM((HIDDEN, 1), jnp.float32),
            pltpu.VMEM((HIDDEN, 1), jnp.float32),
            pltpu.VMEM((HIDDEN, 1), jnp.float32),
            pltpu.VMEM((HIDDEN, HIDDEN), jnp.bfloat16),
            pltpu.VMEM((HIDDEN, 1), jnp.float32),
            pltpu.VMEM((1, HIDDEN), jnp.float32),
            pltpu.VMEM((1, 1), jnp.float32),
        ],
    )

    out_fm = pl.pallas_call(
        _make_body(B, tile_b, needs_mask),
        out_shape=jax.ShapeDtypeStruct((1, padded_b), jnp.float32),
        grid_spec=grid_spec,
        compiler_params=pltpu.CompilerParams(
            dimension_semantics=("arbitrary", "arbitrary")),
    )(x_fm, w1a, w2, b2, w3, b3)

    return out_fm[:, :B].T


# speedup vs baseline: 1.2685x; 1.0816x over previous
import jax
import jax.numpy as jnp
from jax.experimental import pallas as pl
from jax.experimental.pallas import tpu as pltpu

EPS = 1e-5
IN_FEATURES = 8
HIDDEN = 64


def _round_up(n, m):
    return (n + m - 1) // m * m


def _make_body(batch, tile_b, needs_mask):
    inv_b = 1.0 / float(batch)

    def body(x_ref, w1a_ref, w2_ref, b2_ref, w3_ref, b3_ref, o_ref,
             m1_ref, r1_ref, m2_ref, r2_ref,
             w2p_ref, b2p_ref, w3p_ref, b3p_ref, h2c_ref):
        ph = pl.program_id(0)
        t = pl.program_id(1)
        nt = pl.num_programs(1)
        last = nt - 1
        tt = jax.lax.rem(t + 3 * nt - ph, nt)

        def layer1():
            z = jnp.dot(w1a_ref[...], x_ref[...],
                        preferred_element_type=jnp.float32)
            return jnp.maximum(z, 0.0)

        def layer2(h1b):
            z = jnp.dot(w2p_ref[...], h1b, preferred_element_type=jnp.float32)
            return jnp.maximum(z + b2p_ref[...], 0.0)

        def accumulate(h, sum_ref, sq_ref):
            if needs_mask:
                col = (jax.lax.broadcasted_iota(jnp.int32, (1, tile_b), 1)
                       + tt * tile_b)
                h = h * (col < batch).astype(jnp.float32)
            s = jnp.sum(h, axis=1, keepdims=True)
            sq = jnp.sum(h * h, axis=1, keepdims=True)

            @pl.when(t == 0)
            def _():
                sum_ref[...] = s
                sq_ref[...] = sq

            @pl.when(t > 0)
            def _():
                sum_ref[...] += s
                sq_ref[...] += sq

            @pl.when(t == last)
            def _():
                mean = sum_ref[...] * inv_b
                var = sq_ref[...] * inv_b - mean * mean
                sum_ref[...] = mean
                sq_ref[...] = jax.lax.rsqrt(var + EPS)

        @pl.when(ph == 0)
        def _():
            h1 = layer1()
            accumulate(h1, m1_ref, r1_ref)

            @pl.when(t == last)
            def _():
                r1 = r1_ref[...]
                r1_row = r1.reshape(1, HIDDEN)
                w2 = w2_ref[...]
                w2p_ref[...] = (w2 * r1_row).astype(jnp.bfloat16)
                b2p_ref[...] = b2_ref[...] - jnp.dot(
                    w2, m1_ref[...] * r1, preferred_element_type=jnp.float32)

        @pl.when(ph == 1)
        def _():
            def run_phase1(h1b):
                h2 = layer2(h1b)
                accumulate(h2, m2_ref, r2_ref)

                @pl.when(t == last)
                def _():
                    h2c_ref[...] = h2.astype(jnp.bfloat16)
                    r2 = r2_ref[...]
                    w3 = w3_ref[...]
                    w3p_ref[...] = w3 * r2.reshape(1, HIDDEN)
                    b3p_ref[...] = b3_ref[...] - jnp.dot(
                        w3, m2_ref[...] * r2,
                        preferred_element_type=jnp.float32)

            run_phase1(layer1().astype(jnp.bfloat16))

        @pl.when(ph == 2)
        def _():
            w3p = w3p_ref[...]
            b3p = b3p_ref[...]

            @pl.when(t == 0)
            def _():
                out = jnp.dot(w3p.astype(jnp.bfloat16), h2c_ref[...],
                              preferred_element_type=jnp.float32) + b3p
                o_ref[:, pl.ds(tt * tile_b, tile_b)] = out

            @pl.when(t > 0)
            def _():
                h2 = layer2(layer1().astype(jnp.bfloat16))
                out = jnp.dot(w3p, h2,
                              preferred_element_type=jnp.float32) + b3p
                o_ref[:, pl.ds(tt * tile_b, tile_b)] = out

    return body


def kernel(x, w1, b1, w2, b2, w3, b3, *, block_b=131072):
    B, f_in = x.shape
    assert f_in == IN_FEATURES
    assert B > 1

    tile_b = min(_round_up(block_b, 128), _round_up(B, 128))
    padded_b = _round_up(B, tile_b)
    num_tiles = padded_b // tile_b
    needs_mask = padded_b != B

    x_fm = jnp.concatenate(
        [x.astype(jnp.float32).T, jnp.ones((1, B), jnp.float32)],
        axis=0).astype(jnp.bfloat16)
    if needs_mask:
        x_fm = jnp.pad(x_fm, ((0, 0), (0, padded_b - B)))
    w1a = jnp.concatenate([w1, b1], axis=1).astype(jnp.bfloat16)

    def const(ph, t):
        return (0, 0)

    def x_map(ph, t):
        return (0, jax.lax.rem(t + 3 * num_tiles - ph, num_tiles))

    grid_spec = pltpu.PrefetchScalarGridSpec(
        num_scalar_prefetch=0,
        grid=(3, num_tiles),
        in_specs=[
            pl.BlockSpec((IN_FEATURES + 1, tile_b), x_map),
            pl.BlockSpec((HIDDEN, IN_FEATURES + 1), const),
            pl.BlockSpec((HIDDEN, HIDDEN), const),
            pl.BlockSpec((HIDDEN, 1), const),
            pl.BlockSpec((1, HIDDEN), const),
            pl.BlockSpec((1, 1), const),
        ],
        out_specs=pl.BlockSpec((1, padded_b), const),
        scratch_shapes=[
            pltpu.VMEM((HIDDEN, 1), jnp.float32),
            pltpu.VMEM((HIDDEN, 1), jnp.float32),
            pltpu.VMEM((HIDDEN, 1), jnp.float32),
            pltpu.VMEM((HIDDEN, 1), jnp.float32),
            pltpu.VMEM((HIDDEN, HIDDEN), jnp.bfloat16),
            pltpu.VMEM((HIDDEN, 1), jnp.float32),
            pltpu.VMEM((1, HIDDEN), jnp.float32),
            pltpu.VMEM((1, 1), jnp.float32),
            pltpu.VMEM((HIDDEN, tile_b), jnp.bfloat16),
        ],
    )

    out_fm = pl.pallas_call(
        _make_body(B, tile_b, needs_mask),
        out_shape=jax.ShapeDtypeStruct((1, padded_b), jnp.float32),
        grid_spec=grid_spec,
        compiler_params=pltpu.CompilerParams(
            dimension_semantics=("arbitrary", "arbitrary")),
    )(x_fm, w1a, w2, b2, w3, b3)

    return out_fm[:, :B].T
